# Initial kernel scaffold; baseline (speedup 1.0000x reference)
import jax
import jax.numpy as jnp
from jax.experimental import pallas as pl


def kernel(input_ids, W, b):
    raise NotImplementedError("write your pallas kernel here")


# speedup vs baseline: 22.8153x; 22.8153x over previous
import functools

import jax
import jax.numpy as jnp
from jax import lax
from jax.experimental import pallas as pl
from jax.experimental.pallas import tpu as pltpu
from jax.experimental.pallas import tpu_sc as plsc

_VOCAB = 100000
_NUM_CLASSES = 2
_BATCH = 1024
_SEQ = 200
_SEQ_P = 208
_N_SUBCORES = 16
_ROWS_PER = _BATCH // _N_SUBCORES
_IDS_PER = _ROWS_PER * _SEQ_P
_LANES = 16
_GROUPS = _ROWS_PER // _LANES


def _bow_body(ids_hbm, w_hbm, b_hbm, out_hbm, table_v, ids_v, out_v, b_v):
    cls = lax.axis_index("c")
    chunk = lax.axis_index("s")

    w_off = pl.multiple_of(cls * _VOCAB, 8)
    pltpu.sync_copy(w_hbm.at[pl.ds(w_off, _VOCAB)], table_v)
    ids_off = pl.multiple_of(chunk * _IDS_PER, 8)
    pltpu.sync_copy(ids_hbm.at[pl.ds(ids_off, _IDS_PER)], ids_v)
    b_off = pl.multiple_of(cls * _LANES, 8)
    pltpu.sync_copy(b_hbm.at[pl.ds(b_off, _LANES)], b_v)

    lane = lax.iota(jnp.int32, _LANES)
    head = table_v[pl.ds(0, _LANES)]
    table_v[pl.ds(0, _LANES)] = jnp.where(lane == 0, jnp.float32(0.0), head)

    bias = b_v[...]

    for g in range(_GROUPS):
        base = (g * _LANES + lane) * _SEQ_P

        def step(t, acc):
            ids16 = plsc.load_gather(ids_v, [base + t])
            vals = plsc.load_gather(table_v, [ids16])
            return acc + vals

        acc = lax.fori_loop(0, _SEQ_P, step, jnp.zeros((_LANES,), jnp.float32))
        out_v[pl.ds(g * _LANES, _LANES)] = acc + bias

    out_off = pl.multiple_of(cls * _BATCH + chunk * _ROWS_PER, 8)
    pltpu.sync_copy(out_v, out_hbm.at[pl.ds(out_off, _ROWS_PER)])


@jax.jit
def _bow_sc(ids_flat, w_flat, b_bcast):
    mesh = plsc.VectorSubcoreMesh(core_axis_name="c", subcore_axis_name="s")
    f = functools.partial(
        pl.kernel,
        mesh=mesh,
        compiler_params=pltpu.CompilerParams(needs_layout_passes=False),
        out_type=jax.ShapeDtypeStruct((_NUM_CLASSES * _BATCH,), jnp.float32),
        scratch_types=[
            pltpu.VMEM((_VOCAB,), jnp.float32),
            pltpu.VMEM((_IDS_PER,), jnp.int32),
            pltpu.VMEM((_ROWS_PER,), jnp.float32),
            pltpu.VMEM((_LANES,), jnp.float32),
        ],
    )(_bow_body)
    return f(ids_flat, w_flat, b_bcast)


def kernel(input_ids, W, b):
    ids = input_ids.astype(jnp.int32)
    ids_p = jnp.pad(ids, ((0, 0), (0, _SEQ_P - _SEQ))).reshape(-1)
    w_flat = W.astype(jnp.float32).reshape(-1)
    b_bcast = jnp.broadcast_to(b.astype(jnp.float32)[:, None],
                               (_NUM_CLASSES, _LANES)).reshape(-1)
    out = _bow_sc(ids_p, w_flat, b_bcast)
    return out.reshape(_NUM_CLASSES, _BATCH).T


# speedup vs baseline: 27.1429x; 1.1897x over previous
import functools

import jax
import jax.numpy as jnp
from jax import lax
from jax.experimental import pallas as pl
from jax.experimental.pallas import tpu as pltpu
from jax.experimental.pallas import tpu_sc as plsc

_VOCAB = 100000
_NUM_CLASSES = 2
_BATCH = 1024
_SEQ = 200
_N_SUBCORES = 16
_ROWS_PER = _BATCH // _N_SUBCORES
_IDS_PER = _ROWS_PER * _SEQ
_LANES = 16
_GROUPS = _ROWS_PER // _LANES


def _bow_body(ids_hbm, w_hbm, b_hbm, out_hbm, table_v, ids_v, out_v, b_v,
              sem_w, sem_i, sem_b):
    cls = lax.axis_index("c")
    chunk = lax.axis_index("s")

    w_off = pl.multiple_of(cls * _VOCAB, 8)
    cp_w = pltpu.async_copy(w_hbm.at[pl.ds(w_off, _VOCAB)], table_v, sem_w)
    ids_off = pl.multiple_of(chunk * _IDS_PER, 8)
    cp_i = pltpu.async_copy(ids_hbm.at[pl.ds(ids_off, _IDS_PER)], ids_v, sem_i)
    b_off = pl.multiple_of(cls * _LANES, 8)
    cp_b = pltpu.async_copy(b_hbm.at[pl.ds(b_off, _LANES)], b_v, sem_b)
    cp_i.wait()
    cp_b.wait()
    cp_w.wait()

    lane = lax.iota(jnp.int32, _LANES)
    head = table_v[pl.ds(0, _LANES)]
    table_v[pl.ds(0, _LANES)] = jnp.where(lane == 0, jnp.float32(0.0), head)

    bias = b_v[...]
    bases = [(g * _LANES + lane) * _SEQ for g in range(_GROUPS)]
    zero = jnp.zeros((_LANES,), jnp.float32)

    def step(t, accs):
        ids16 = [plsc.load_gather(ids_v, [bases[g] + t])
                 for g in range(_GROUPS)]
        vals = [plsc.load_gather(table_v, [ids16[g]]) for g in range(_GROUPS)]
        return tuple(accs[g] + vals[g] for g in range(_GROUPS))

    accs = lax.fori_loop(0, _SEQ, step, (zero,) * _GROUPS)
    for g in range(_GROUPS):
        out_v[pl.ds(g * _LANES, _LANES)] = accs[g] + bias

    out_off = pl.multiple_of(cls * _BATCH + chunk * _ROWS_PER, 8)
    pltpu.sync_copy(out_v, out_hbm.at[pl.ds(out_off, _ROWS_PER)])


@jax.jit
def _bow_sc(ids_flat, w_flat, b_bcast):
    mesh = plsc.VectorSubcoreMesh(core_axis_name="c", subcore_axis_name="s")
    f = functools.partial(
        pl.kernel,
        mesh=mesh,
        compiler_params=pltpu.CompilerParams(needs_layout_passes=False),
        out_type=jax.ShapeDtypeStruct((_NUM_CLASSES * _BATCH,), jnp.float32),
        scratch_types=[
            pltpu.VMEM((_VOCAB,), jnp.float32),
            pltpu.VMEM((_IDS_PER,), jnp.int32),
            pltpu.VMEM((_ROWS_PER,), jnp.float32),
            pltpu.VMEM((_LANES,), jnp.float32),
            pltpu.SemaphoreType.DMA,
            pltpu.SemaphoreType.DMA,
            pltpu.SemaphoreType.DMA,
        ],
    )(_bow_body)
    return f(ids_flat, w_flat, b_bcast)


def kernel(input_ids, W, b):
    ids_flat = input_ids.astype(jnp.int32).reshape(-1)
    w_flat = W.astype(jnp.float32).reshape(-1)
    b_bcast = jnp.broadcast_to(b.astype(jnp.float32)[:, None],
                               (_NUM_CLASSES, _LANES)).reshape(-1)
    out = _bow_sc(ids_flat, w_flat, b_bcast)
    return out.reshape(_NUM_CLASSES, _BATCH).T
